# Initial kernel scaffold; baseline (speedup 1.0000x reference)
import functools

import jax
import jax.numpy as jnp
from jax import lax
from jax.experimental import pallas as pl
from jax.experimental.pallas import tpu as pltpu
from jax.experimental.pallas import tpu_sc as plsc

B, L = 16384, 200
N = B * L
NC, NS = 2, 16
NW = NC * NS
PER_W = N // NW
S = 5120
ITERS = PER_W // S
GROUPS = S // 16


def _sc_body(dow_hbm, month_hbm, opex_hbm, qtr_hbm, dtab_hbm, mtab_hbm,
             out_hbm, dtab_v, mtab_v, dow_v, month_v, opex_v, qtr_v, out_v):
    wid = lax.axis_index("s") * NC + lax.axis_index("c")
    pltpu.sync_copy(dtab_hbm, dtab_v)
    pltpu.sync_copy(mtab_hbm, mtab_v)
    iota10 = lax.iota(jnp.int32, 16) * 10

    def chunk(it, _):
        base = wid * PER_W + it * S
        pltpu.sync_copy(dow_hbm.at[pl.ds(base, S)], dow_v)
        pltpu.sync_copy(month_hbm.at[pl.ds(base, S)], month_v)
        pltpu.sync_copy(opex_hbm.at[pl.ds(base, S)], opex_v)
        pltpu.sync_copy(qtr_hbm.at[pl.ds(base, S)], qtr_v)

        def grp(g, _):
            p = g * 16
            d4 = dow_v[pl.ds(p, 16)] * 4
            m4 = month_v[pl.ds(p, 16)] * 4
            off = iota10 + g * 160
            for c in range(4):
                plsc.store_scatter(out_v, [off + c],
                                   plsc.load_gather(dtab_v, [d4 + c]))
            for c in range(4):
                plsc.store_scatter(out_v, [off + (4 + c)],
                                   plsc.load_gather(mtab_v, [m4 + c]))
            plsc.store_scatter(out_v, [off + 8], opex_v[pl.ds(p, 16)])
            plsc.store_scatter(out_v, [off + 9], qtr_v[pl.ds(p, 16)])
            return 0

        lax.fori_loop(0, GROUPS, grp, 0)
        pltpu.sync_copy(out_v, out_hbm.at[pl.ds(base * 10, S * 10)])
        return 0

    lax.fori_loop(0, ITERS, chunk, 0)


@jax.jit
def _run(dow_f, month_f, opex_f, qtr_f, dtab, mtab):
    mesh = plsc.VectorSubcoreMesh(core_axis_name="c", subcore_axis_name="s",
                                  num_cores=NC, num_subcores=NS)
    f = pl.kernel(
        _sc_body,
        out_type=jax.ShapeDtypeStruct((N * 10,), jnp.float32),
        mesh=mesh,
        scratch_types=[
            pltpu.VMEM((24,), jnp.float32),
            pltpu.VMEM((48,), jnp.float32),
            pltpu.VMEM((S,), jnp.int32),
            pltpu.VMEM((S,), jnp.int32),
            pltpu.VMEM((S,), jnp.float32),
            pltpu.VMEM((S,), jnp.float32),
            pltpu.VMEM((S * 10,), jnp.float32),
        ],
    )
    return f(dow_f, month_f, opex_f, qtr_f, dtab, mtab)


def kernel(dow, month, is_opex, is_qtr_end, dow_table, month_table):
    dow_f = dow.reshape(N).astype(jnp.int32)
    month_f = month.reshape(N).astype(jnp.int32)
    opex_f = is_opex.reshape(N)
    qtr_f = is_qtr_end.reshape(N)
    dtab = jnp.pad(dow_table.reshape(20), (0, 4))
    mtab = month_table.reshape(48)
    out = _run(dow_f, month_f, opex_f, qtr_f, dtab, mtab)
    return out.reshape(B, L, 10)


# speedup vs baseline: 7.9872x; 7.9872x over previous
import functools

import jax
import jax.numpy as jnp
from jax import lax
from jax.experimental import pallas as pl
from jax.experimental.pallas import tpu as pltpu
from jax.experimental.pallas import tpu_sc as plsc

B, L = 16384, 200
N = B * L
NC, NS = 2, 16
NW = NC * NS
PER_W = N // NW
S = 5120
ITERS = PER_W // S
GROUPS = S // 16


def _sc_body(dow_hbm, month_hbm, opex_hbm, qtr_hbm, dtab_hbm, mtab_hbm,
             out_hbm, dtab_v, mtab_v, dow_v, month_v, opex_v, qtr_v, out_v):
    wid = lax.axis_index("s") * NC + lax.axis_index("c")
    pltpu.sync_copy(dtab_hbm, dtab_v)
    pltpu.sync_copy(mtab_hbm, mtab_v)
    iota10 = lax.iota(jnp.int32, 16) * 10

    def chunk(it, _):
        base = wid * PER_W + it * S
        pltpu.sync_copy(dow_hbm.at[pl.ds(base, S)], dow_v)
        pltpu.sync_copy(month_hbm.at[pl.ds(base, S)], month_v)
        pltpu.sync_copy(opex_hbm.at[pl.ds(base, S)], opex_v)
        pltpu.sync_copy(qtr_hbm.at[pl.ds(base, S)], qtr_v)

        def grp(g, _):
            p = g * 16
            d4 = dow_v[pl.ds(p, 16)] * 4
            m4 = month_v[pl.ds(p, 16)] * 4
            off = iota10 + g * 160
            for c in range(4):
                plsc.store_scatter(out_v, [off + c],
                                   plsc.load_gather(dtab_v, [d4 + c]))
            for c in range(4):
                plsc.store_scatter(out_v, [off + (4 + c)],
                                   plsc.load_gather(mtab_v, [m4 + c]))
            plsc.store_scatter(out_v, [off + 8], opex_v[pl.ds(p, 16)])
            plsc.store_scatter(out_v, [off + 9], qtr_v[pl.ds(p, 16)])
            return 0

        lax.fori_loop(0, GROUPS, grp, 0)
        pltpu.sync_copy(out_v, out_hbm.at[pl.ds(base * 10, S * 10)])
        return 0

    lax.fori_loop(0, ITERS, chunk, 0)


@jax.jit
def _run(dow_f, month_f, opex_f, qtr_f, dtab, mtab):
    mesh = plsc.VectorSubcoreMesh(core_axis_name="c", subcore_axis_name="s",
                                  num_cores=NC, num_subcores=NS)
    f = pl.kernel(
        _sc_body,
        out_type=jax.ShapeDtypeStruct((N * 10,), jnp.float32),
        mesh=mesh,
        scratch_types=[
            pltpu.VMEM((24,), jnp.float32),
            pltpu.VMEM((48,), jnp.float32),
            pltpu.VMEM((S,), jnp.int32),
            pltpu.VMEM((S,), jnp.int32),
            pltpu.VMEM((S,), jnp.float32),
            pltpu.VMEM((S,), jnp.float32),
            pltpu.VMEM((S * 10,), jnp.float32),
        ],
        compiler_params=pltpu.CompilerParams(needs_layout_passes=False),
    )
    return f(dow_f, month_f, opex_f, qtr_f, dtab, mtab)


def kernel(dow, month, is_opex, is_qtr_end, dow_table, month_table):
    dow_f = dow.reshape(N).astype(jnp.int32)
    month_f = month.reshape(N).astype(jnp.int32)
    opex_f = is_opex.reshape(N)
    qtr_f = is_qtr_end.reshape(N)
    dtab = jnp.pad(dow_table.reshape(20), (0, 4))
    mtab = month_table.reshape(48)
    out = _run(dow_f, month_f, opex_f, qtr_f, dtab, mtab)
    return out.reshape(B, L, 10)


# speedup vs baseline: 8.6934x; 1.0884x over previous
import functools

import jax
import jax.numpy as jnp
from jax import lax
from jax.experimental import pallas as pl
from jax.experimental.pallas import tpu as pltpu
from jax.experimental.pallas import tpu_sc as plsc

B, L = 16384, 200
N = B * L
NC, NS = 2, 16
NW = NC * NS
PER_W = N // NW
S = 5120
ITERS = PER_W // S
GROUPS = S // 16


def _sc_body(dow_hbm, month_hbm, opex_hbm, qtr_hbm, dtab_hbm, mtab_hbm,
             out_hbm, dtab_v, mtab_v, dow_v, month_v, opex_v, qtr_v, out_v):
    wid = lax.axis_index("s") * NC + lax.axis_index("c")
    pltpu.sync_copy(dtab_hbm, dtab_v)
    pltpu.sync_copy(mtab_hbm, mtab_v)
    iota10 = lax.iota(jnp.int32, 16) * 10

    def chunk(it, _):
        base = wid * PER_W + it * S
        pltpu.sync_copy(dow_hbm.at[pl.ds(base, S)], dow_v)
        pltpu.sync_copy(month_hbm.at[pl.ds(base, S)], month_v)
        pltpu.sync_copy(opex_hbm.at[pl.ds(base, S)], opex_v)
        pltpu.sync_copy(qtr_hbm.at[pl.ds(base, S)], qtr_v)

        @plsc.parallel_loop(0, GROUPS, unroll=8)
        def grp(g):
            p = g * 16
            d4 = dow_v[pl.ds(p, 16)] * 4
            m4 = month_v[pl.ds(p, 16)] * 4
            off = iota10 + g * 160
            for c in range(4):
                plsc.store_scatter(out_v, [off + c],
                                   plsc.load_gather(dtab_v, [d4 + c]))
            for c in range(4):
                plsc.store_scatter(out_v, [off + (4 + c)],
                                   plsc.load_gather(mtab_v, [m4 + c]))
            plsc.store_scatter(out_v, [off + 8], opex_v[pl.ds(p, 16)])
            plsc.store_scatter(out_v, [off + 9], qtr_v[pl.ds(p, 16)])
        pltpu.sync_copy(out_v, out_hbm.at[pl.ds(base * 10, S * 10)])
        return 0

    lax.fori_loop(0, ITERS, chunk, 0)


@jax.jit
def _run(dow_f, month_f, opex_f, qtr_f, dtab, mtab):
    mesh = plsc.VectorSubcoreMesh(core_axis_name="c", subcore_axis_name="s",
                                  num_cores=NC, num_subcores=NS)
    f = pl.kernel(
        _sc_body,
        out_type=jax.ShapeDtypeStruct((N * 10,), jnp.float32),
        mesh=mesh,
        scratch_types=[
            pltpu.VMEM((24,), jnp.float32),
            pltpu.VMEM((48,), jnp.float32),
            pltpu.VMEM((S,), jnp.int32),
            pltpu.VMEM((S,), jnp.int32),
            pltpu.VMEM((S,), jnp.float32),
            pltpu.VMEM((S,), jnp.float32),
            pltpu.VMEM((S * 10,), jnp.float32),
        ],
        compiler_params=pltpu.CompilerParams(needs_layout_passes=False),
    )
    return f(dow_f, month_f, opex_f, qtr_f, dtab, mtab)


def kernel(dow, month, is_opex, is_qtr_end, dow_table, month_table):
    dow_f = dow.reshape(N).astype(jnp.int32)
    month_f = month.reshape(N).astype(jnp.int32)
    opex_f = is_opex.reshape(N)
    qtr_f = is_qtr_end.reshape(N)
    dtab = jnp.pad(dow_table.reshape(20), (0, 4))
    mtab = month_table.reshape(48)
    out = _run(dow_f, month_f, opex_f, qtr_f, dtab, mtab)
    return out.reshape(B, L, 10)


# speedup vs baseline: 116.8011x; 13.4356x over previous
import jax
import jax.numpy as jnp
from jax import lax
from jax.experimental import pallas as pl
from jax.experimental.pallas import tpu as pltpu
from jax.experimental.pallas import tpu_sc as plsc

B, L = 16384, 200
N = B * L
NC, NS = 2, 16
NW = NC * NS
S = 4096
CPR = B // S
NCHUNK = N // S
PER_W = NCHUNK // NW
GROUPS = S // 16


def _sc_body(dow_hbm, month_hbm, opex_hbm, qtr_hbm, dtab_hbm, mtab_hbm,
             out_hbm, dtab_v, mtab_v, dow_v, month_v, opex_v, qtr_v, oc_v):
    wid = lax.axis_index("s") * NC + lax.axis_index("c")
    pltpu.sync_copy(dtab_hbm, dtab_v)
    pltpu.sync_copy(mtab_hbm, mtab_v)

    def chunk(t, _):
        cid = wid * PER_W + t
        r = cid // CPR
        b0 = (cid % CPR) * S
        pltpu.sync_copy(dow_hbm.at[r, pl.ds(b0, S)], dow_v)
        pltpu.sync_copy(month_hbm.at[r, pl.ds(b0, S)], month_v)
        pltpu.sync_copy(opex_hbm.at[r, pl.ds(b0, S)], opex_v)
        pltpu.sync_copy(qtr_hbm.at[r, pl.ds(b0, S)], qtr_v)

        @plsc.parallel_loop(0, GROUPS, unroll=8)
        def grp(g):
            p = g * 16
            d4 = dow_v[pl.ds(p, 16)] * 4
            m4 = month_v[pl.ds(p, 16)] * 4
            for c in range(4):
                oc_v[c, pl.ds(p, 16)] = plsc.load_gather(dtab_v, [d4 + c])
            for c in range(4):
                oc_v[4 + c, pl.ds(p, 16)] = plsc.load_gather(mtab_v, [m4 + c])

        for c in range(8):
            pltpu.sync_copy(oc_v.at[c], out_hbm.at[c, r, pl.ds(b0, S)])
        pltpu.sync_copy(opex_v, out_hbm.at[8, r, pl.ds(b0, S)])
        pltpu.sync_copy(qtr_v, out_hbm.at[9, r, pl.ds(b0, S)])
        return 0

    lax.fori_loop(0, PER_W, chunk, 0)


@jax.jit
def _run(dow_t, month_t, opex_t, qtr_t, dtab, mtab):
    mesh = plsc.VectorSubcoreMesh(core_axis_name="c", subcore_axis_name="s",
                                  num_cores=NC, num_subcores=NS)
    f = pl.kernel(
        _sc_body,
        out_type=jax.ShapeDtypeStruct((10, L, B), jnp.float32),
        mesh=mesh,
        scratch_types=[
            pltpu.VMEM((24,), jnp.float32),
            pltpu.VMEM((48,), jnp.float32),
            pltpu.VMEM((S,), jnp.int32),
            pltpu.VMEM((S,), jnp.int32),
            pltpu.VMEM((S,), jnp.float32),
            pltpu.VMEM((S,), jnp.float32),
            pltpu.VMEM((8, S), jnp.float32),
        ],
        compiler_params=pltpu.CompilerParams(needs_layout_passes=False),
    )
    return f(dow_t, month_t, opex_t, qtr_t, dtab, mtab)


def kernel(dow, month, is_opex, is_qtr_end, dow_table, month_table):
    dow_t = dow.T.astype(jnp.int32)
    month_t = month.T.astype(jnp.int32)
    dtab = jnp.pad(dow_table.reshape(20), (0, 4))
    mtab = month_table.reshape(48)
    out = _run(dow_t, month_t, is_opex.T, is_qtr_end.T, dtab, mtab)
    return out.transpose(2, 1, 0)
